# baseline (speedup 1.0000x reference)
import functools

import jax
import jax.numpy as jnp
from jax import lax
from jax.experimental import pallas as pl
from jax.experimental.pallas import tpu as pltpu
from jax.experimental.pallas import tpu_sc as plsc

N_DRUG = 10000
N_DIS = 10000
N_ALL = N_DRUG + N_DIS
E = 320000
H = 128

NC = 2
NS = 16
L = 16

HIST = 20480
_DEG_C = 80
_DEG_ROWS = 16384
_DEG_ROWS_SUB = _DEG_ROWS // (NC * NS)
_DEG_FIRE = 8
_HIST_PER_SUB = HIST // NS


def _sc_degree_body(idx_hbm, out_hbm, hist_sp, idx_blk, ones_v, zero_v, sem):
    c = lax.axis_index("c")
    s = lax.axis_index("s")
    zeros16 = jnp.zeros((L,), jnp.float32)
    ones16 = jnp.ones((L,), jnp.float32)
    def fill_zero(i, _):
        zero_v[pl.ds(i * L, L)] = zeros16
        return 0
    lax.fori_loop(0, _HIST_PER_SUB // L, fill_zero, 0)
    for k in range(_DEG_C // L):
        ones_v[pl.ds(k * L, L)] = ones16
    pltpu.sync_copy(zero_v, hist_sp.at[pl.ds(s * _HIST_PER_SUB, _HIST_PER_SUB)])

    row0 = c * (_DEG_ROWS // 2) + s * _DEG_ROWS_SUB
    pltpu.sync_copy(idx_hbm.at[pl.ds(row0, _DEG_ROWS_SUB)], idx_blk)
    plsc.subcore_barrier()

    def body(g, _):
        for k in range(_DEG_FIRE):
            pltpu.async_copy(ones_v, hist_sp.at[idx_blk.at[g * _DEG_FIRE + k]],
                             sem, add=True)
        for k in range(_DEG_FIRE):
            pltpu.make_async_copy(ones_v, hist_sp.at[idx_blk.at[0]], sem).wait()
        return 0

    lax.fori_loop(0, _DEG_ROWS_SUB // _DEG_FIRE, body, 0)
    plsc.subcore_barrier()
    pltpu.sync_copy(hist_sp.at[pl.ds(s * _HIST_PER_SUB, _HIST_PER_SUB)],
                    out_hbm.at[c, pl.ds(s * _HIST_PER_SUB, _HIST_PER_SUB)])


def _sc_degrees(idx_blocks):
    mesh = plsc.VectorSubcoreMesh(core_axis_name="c", subcore_axis_name="s",
                                  num_cores=NC, num_subcores=NS)
    return pl.kernel(
        _sc_degree_body,
        out_type=jax.ShapeDtypeStruct((NC, HIST), jnp.float32),
        mesh=mesh,
        scratch_types=[
            pltpu.VMEM_SHARED((HIST,), jnp.float32),
            pltpu.VMEM((_DEG_ROWS_SUB, _DEG_C), jnp.int32),
            pltpu.VMEM((_DEG_C,), jnp.float32),
            pltpu.VMEM((_HIST_PER_SUB,), jnp.float32),
            pltpu.SemaphoreType.DMA,
        ],
    )(idx_blocks)


_AGG_C = 104
_AGG_CHUNKS = 193
N_PAD = 10112
_ROWS_PER_SUB = N_PAD // NS
_ZBLK = 8


def _sc_agg_body(table_hbm, src_hbm, dst_hbm, out_hbm,
                 acc_sp, sidx, didx, rows,
                 g0, g1, g2, s0, s1, s2, i0, i1, i2):
    c = lax.axis_index("c")
    s = lax.axis_index("s")
    zeros16 = jnp.zeros((L,), jnp.float32)
    for r in range(_ZBLK):
        for k in range(H // L):
            rows[0, r, pl.ds(k * L, L)] = zeros16

    row0 = s * _ROWS_PER_SUB
    zsrc = rows.at[0].at[pl.ds(0, _ZBLK)]

    def zbody(i, _):
        pltpu.sync_copy(zsrc, acc_sp.at[pl.ds(row0 + i * _ZBLK, _ZBLK)])
        return 0

    lax.fori_loop(0, _ROWS_PER_SUB // _ZBLK, zbody, 0)
    plsc.subcore_barrier()

    gsems = (g0, g1, g2)
    ssems = (s0, s1, s2)
    isems = (i0, i1, i2)

    def idx_start(j, b):
        pltpu.async_copy(src_hbm.at[c, s, j], sidx.at[b], isems[b])
        pltpu.async_copy(dst_hbm.at[c, s, j], didx.at[b], isems[b])

    def idx_wait(j, b):
        pltpu.make_async_copy(src_hbm.at[c, s, j], sidx.at[b], isems[b]).wait()
        pltpu.make_async_copy(dst_hbm.at[c, s, j], didx.at[b], isems[b]).wait()

    idx_start(0, 0)
    idx_start(1, 1)
    idx_wait(0, 0)
    pltpu.async_copy(table_hbm.at[sidx.at[0]], rows.at[0], gsems[0])

    def group(g, _):
        for b in range(3):
            bn = (b + 1) % 3
            bp = (b - 1) % 3
            j = 3 * g + b
            nxt = j + 1

            @pl.when(nxt < _AGG_CHUNKS)
            def _():
                idx_wait(nxt, bn)
                pltpu.async_copy(table_hbm.at[sidx.at[bn]], rows.at[bn],
                                 gsems[bn])
            pltpu.make_async_copy(table_hbm.at[sidx.at[b]], rows.at[b],
                                  gsems[b]).wait()
            pltpu.async_copy(rows.at[b], acc_sp.at[didx.at[b]], ssems[b],
                             add=True)

            @pl.when(j >= 1)
            def _():
                pltpu.make_async_copy(rows.at[bp], acc_sp.at[didx.at[bp]],
                                      ssems[bp]).wait()

            @pl.when(nxt + 1 < _AGG_CHUNKS)
            def _():
                idx_start(nxt + 1, bp)
        return 0

    lax.fori_loop(0, _AGG_CHUNKS // 3, group, 0)
    pltpu.make_async_copy(table_hbm.at[sidx.at[0]], rows.at[0],
                          gsems[0]).wait()
    pltpu.async_copy(rows.at[0], acc_sp.at[didx.at[0]], ssems[0], add=True)
    pltpu.make_async_copy(rows.at[2], acc_sp.at[didx.at[2]], ssems[2]).wait()
    pltpu.make_async_copy(rows.at[0], acc_sp.at[didx.at[0]], ssems[0]).wait()

    plsc.subcore_barrier()
    pltpu.sync_copy(acc_sp.at[pl.ds(row0, _ROWS_PER_SUB)],
                    out_hbm.at[c, pl.ds(row0, _ROWS_PER_SUB)])


def _sc_aggregate(table, src_blocks, dst_blocks):
    mesh = plsc.VectorSubcoreMesh(core_axis_name="c", subcore_axis_name="s",
                                  num_cores=NC, num_subcores=NS)
    return pl.kernel(
        _sc_agg_body,
        out_type=jax.ShapeDtypeStruct((NC, N_PAD, H), jnp.float32),
        mesh=mesh,
        scratch_types=[
            pltpu.VMEM_SHARED((N_PAD, H), jnp.float32),
            pltpu.VMEM((3, _AGG_C), jnp.int32),
            pltpu.VMEM((3, _AGG_C), jnp.int32),
            pltpu.VMEM((3, _AGG_C, H), jnp.float32),
        ] + [pltpu.SemaphoreType.DMA] * 9,
    )(table, src_blocks, dst_blocks)


_BM = 1000


def _rows_call(body, n_out, *args):
    in_specs = []
    for a in args:
        if a.shape[0] == N_DRUG:
            in_specs.append(pl.BlockSpec((_BM, a.shape[1]), lambda i: (i, 0)))
        else:
            in_specs.append(pl.BlockSpec(a.shape, lambda i: (0, 0)))
    outs = [jax.ShapeDtypeStruct((N_DRUG, H), jnp.float32)] * n_out
    return pl.pallas_call(
        body,
        grid=(N_DRUG // _BM,),
        in_specs=in_specs,
        out_specs=[pl.BlockSpec((_BM, H), lambda i: (i, 0))] * n_out,
        out_shape=outs,
    )(*args)


def _proj_body(x_ref, W_ref, b_ref, degs_ref, h_ref, tab_ref):
    h = lax.dot_general(x_ref[...], W_ref[...], (((1,), (1,)), ((), ())),
                        preferred_element_type=jnp.float32) + b_ref[...]
    h_ref[...] = h
    tab_ref[...] = h * lax.rsqrt(jnp.maximum(degs_ref[...], 1.0))


def _post_body(agg_ref, degd_ref, degs_ref, W_ref, b_ref, gam_ref, bet_ref,
               a_ref, h_ref, tab_ref):
    x = agg_ref[...] * lax.rsqrt(jnp.maximum(degd_ref[...], 1.0))
    v = lax.dot_general(x, W_ref[...], (((1,), (1,)), ((), ())),
                        preferred_element_type=jnp.float32) + b_ref[...]
    v = gam_ref[...] * v + bet_ref[...]
    h = jnp.where(v >= 0, v, a_ref[...] * v)
    h_ref[...] = h
    tab_ref[...] = h * lax.rsqrt(jnp.maximum(degs_ref[...], 1.0))


def _att_score_body(h0_ref, h1_ref, h2_ref, W1_ref, b1_ref, w2_ref, o_ref):
    step = pl.program_id(0)

    @pl.when(step == 0)
    def _():
        o_ref[...] = jnp.zeros_like(o_ref)

    lane = lax.broadcasted_iota(jnp.int32, (1, H), 1)
    acc = o_ref[...]
    for l, h_ref in enumerate((h0_ref, h1_ref, h2_ref)):
        t = jnp.tanh(lax.dot_general(h_ref[...], W1_ref[...],
                                     (((1,), (1,)), ((), ())),
                                     preferred_element_type=jnp.float32)
                     + b1_ref[...])
        sl = jnp.sum(t * w2_ref[...])
        acc = acc + jnp.where(lane == l, sl, 0.0)
    o_ref[...] = acc


def _att_scores(h0, h1, h2, W1, b1, w2):
    return pl.pallas_call(
        _att_score_body,
        grid=(N_DRUG // _BM,),
        in_specs=[pl.BlockSpec((_BM, H), lambda i: (i, 0))] * 3
        + [pl.BlockSpec((H, H), lambda i: (0, 0)),
           pl.BlockSpec((1, H), lambda i: (0, 0)),
           pl.BlockSpec((1, H), lambda i: (0, 0))],
        out_specs=pl.BlockSpec((1, H), lambda i: (0, 0)),
        out_shape=jax.ShapeDtypeStruct((1, H), jnp.float32),
    )(h0, h1, h2, W1, b1, w2)


def _combine_body(h0_ref, h1_ref, h2_ref, beta_ref, wr_ref, wd_ref, o_ref):
    f = (h0_ref[...] * beta_ref[0, 0] + h1_ref[...] * beta_ref[0, 1]
         + h2_ref[...] * beta_ref[0, 2])
    m = lax.dot_general(wr_ref[...], wd_ref[...], (((0,), (0,)), ((), ())),
                        preferred_element_type=jnp.float32)
    o_ref[...] = jnp.dot(f, m,
                         preferred_element_type=jnp.float32).astype(jnp.bfloat16)


def _combine(h0, h1, h2, beta_pad, wr, wd):
    return pl.pallas_call(
        _combine_body,
        grid=(N_DRUG // _BM,),
        in_specs=[pl.BlockSpec((_BM, H), lambda i: (i, 0))] * 3
        + [pl.BlockSpec((1, H), lambda i: (0, 0)),
           pl.BlockSpec((H, H), lambda i: (0, 0)),
           pl.BlockSpec((H, H), lambda i: (0, 0))],
        out_specs=pl.BlockSpec((_BM, H), lambda i: (i, 0)),
        out_shape=jax.ShapeDtypeStruct((N_DRUG, H), jnp.bfloat16),
    )(h0, h1, h2, beta_pad, wr, wd)


def _decoder_matmul_kernel(a_ref, b_ref, o_ref):
    o_ref[...] = lax.dot_general(
        a_ref[...], b_ref[...], (((1,), (1,)), ((), ())),
        preferred_element_type=jnp.float32)


def _decoder_matmul(a, b, bm=512, bn=512):
    m, k = a.shape
    n = b.shape[0]
    grid = (pl.cdiv(m, bm), pl.cdiv(n, bn))
    return pl.pallas_call(
        _decoder_matmul_kernel,
        grid=grid,
        in_specs=[
            pl.BlockSpec((bm, k), lambda i, j: (i, 0)),
            pl.BlockSpec((bn, k), lambda i, j: (j, 0)),
        ],
        out_specs=pl.BlockSpec((bm, bn), lambda i, j: (i, j)),
        out_shape=jax.ShapeDtypeStruct((m, n), jnp.float32),
    )(a, b)


def _edge_blocks(idx, offset, pad_value):
    per_sub = E // NS
    pad = _AGG_CHUNKS * _AGG_C - per_sub
    blk = idx.reshape(NS, per_sub) + offset
    blk = jnp.pad(blk, ((0, 0), (0, pad)), constant_values=pad_value)
    return blk.reshape(NS, _AGG_CHUNKS, _AGG_C)


def kernel(x_drug, x_disease, edge_dr2di, edge_di2dr,
           W_drug_lin, b_drug_lin, W_dis_lin, b_dis_lin,
           e1_W_dr2di, e1_b_dr2di, e1_W_di2dr, e1_b_di2dr, e1_gamma, e1_beta, e1_prelu,
           e2_W_dr2di, e2_b_dr2di, e2_W_di2dr, e2_b_di2dr, e2_gamma, e2_beta, e2_prelu,
           att_dr_W1, att_dr_b1, att_dr_w2, att_di_W1, att_di_b1, att_di_w2,
           W_R, W_D):
    src_blocks = jnp.stack([
        _edge_blocks(edge_dr2di[0], 0, 0),
        _edge_blocks(edge_di2dr[0], N_DRUG, 0),
    ])
    dst_blocks = jnp.stack([
        _edge_blocks(edge_dr2di[1], 0, N_PAD - 8),
        _edge_blocks(edge_di2dr[1], 0, N_PAD - 8),
    ])

    src_all = jnp.concatenate([edge_dr2di[0], edge_di2dr[0] + N_DRUG])
    dst_off = jnp.concatenate([edge_dr2di[1], edge_di2dr[1] + N_DIS])
    deg_idx = jnp.concatenate([src_all, dst_off])
    deg_pad = _DEG_ROWS * _DEG_C - deg_idx.shape[0]
    deg_idx = jnp.pad(deg_idx, (0, deg_pad), constant_values=N_ALL)
    deg_blocks = deg_idx.reshape(_DEG_ROWS, _DEG_C)

    hists = _sc_degrees(deg_blocks)
    deg_s_dr = hists[0, :N_DRUG].reshape(-1, 1)
    deg_s_di = hists[0, N_DRUG:N_ALL].reshape(-1, 1)
    deg_d_di = hists[1, :N_DIS].reshape(-1, 1)
    deg_d_dr = hists[1, N_DIS:N_ALL].reshape(-1, 1)

    h_dr0, tab_dr = _rows_call(_proj_body, 2, x_drug, W_drug_lin,
                               b_drug_lin.reshape(1, -1), deg_s_dr)
    h_di0, tab_di = _rows_call(_proj_body, 2, x_disease, W_dis_lin,
                               b_dis_lin.reshape(1, -1), deg_s_di)

    g1 = e1_gamma.reshape(1, -1)
    be1 = e1_beta.reshape(1, -1)
    a1 = jnp.broadcast_to(e1_prelu.reshape(1, 1), (1, H))
    g2 = e2_gamma.reshape(1, -1)
    be2 = e2_beta.reshape(1, -1)
    a2 = jnp.broadcast_to(e2_prelu.reshape(1, 1), (1, H))

    table1 = jnp.concatenate([tab_dr, tab_di])
    agg1 = _sc_aggregate(table1, src_blocks, dst_blocks)
    h_di1, tab_di1 = _rows_call(_post_body, 2, agg1[0, :N_DIS], deg_d_di,
                                deg_s_di, e1_W_dr2di,
                                e1_b_dr2di.reshape(1, -1), g1, be1, a1)
    h_dr1, tab_dr1 = _rows_call(_post_body, 2, agg1[1, :N_DRUG], deg_d_dr,
                                deg_s_dr, e1_W_di2dr,
                                e1_b_di2dr.reshape(1, -1), g1, be1, a1)

    table2 = jnp.concatenate([tab_dr1, tab_di1])
    agg2 = _sc_aggregate(table2, src_blocks, dst_blocks)
    h_di2, _ = _rows_call(_post_body, 2, agg2[0, :N_DIS], deg_d_di,
                          deg_s_di, e2_W_dr2di,
                          e2_b_dr2di.reshape(1, -1), g2, be2, a2)
    h_dr2, _ = _rows_call(_post_body, 2, agg2[1, :N_DRUG], deg_d_dr,
                          deg_s_dr, e2_W_di2dr,
                          e2_b_di2dr.reshape(1, -1), g2, be2, a2)

    s_dr = _att_scores(h_dr0, h_dr1, h_dr2, att_dr_W1,
                       att_dr_b1.reshape(1, -1), att_dr_w2.reshape(1, -1))
    s_di = _att_scores(h_di0, h_di1, h_di2, att_di_W1,
                       att_di_b1.reshape(1, -1), att_di_w2.reshape(1, -1))
    beta_dr = jnp.pad(jax.nn.softmax(s_dr[0, :3] / N_DRUG), (0, H - 3))
    beta_di = jnp.pad(jax.nn.softmax(s_di[0, :3] / N_DIS), (0, H - 3))

    eye = jnp.eye(H, dtype=jnp.float32)
    a_mat = _combine(h_dr0, h_dr1, h_dr2, beta_dr.reshape(1, H), W_R, W_D)
    b_mat = _combine(h_di0, h_di1, h_di2, beta_di.reshape(1, H), eye, eye)
    return _decoder_matmul(a_mat, b_mat)


# speedup vs baseline: 1.0868x; 1.0868x over previous
import functools

import jax
import jax.numpy as jnp
from jax import lax
from jax.experimental import pallas as pl
from jax.experimental.pallas import tpu as pltpu
from jax.experimental.pallas import tpu_sc as plsc

N_DRUG = 10000
N_DIS = 10000
N_ALL = N_DRUG + N_DIS
E = 320000
H = 128

NC = 2
NS = 16
L = 16

HIST = 20480
_DEG_C = 80
_DEG_ROWS = 16384
_DEG_ROWS_SUB = _DEG_ROWS // (NC * NS)
_DEG_FIRE = 8
_HIST_PER_SUB = HIST // NS


def _sc_degree_body(idx_hbm, out_hbm, hist_sp, idx_blk, ones_v, zero_v, sem):
    c = lax.axis_index("c")
    s = lax.axis_index("s")
    zeros16 = jnp.zeros((L,), jnp.float32)
    ones16 = jnp.ones((L,), jnp.float32)
    def fill_zero(i, _):
        zero_v[pl.ds(i * L, L)] = zeros16
        return 0
    lax.fori_loop(0, _HIST_PER_SUB // L, fill_zero, 0)
    for k in range(_DEG_C // L):
        ones_v[pl.ds(k * L, L)] = ones16
    pltpu.sync_copy(zero_v, hist_sp.at[pl.ds(s * _HIST_PER_SUB, _HIST_PER_SUB)])

    row0 = c * (_DEG_ROWS // 2) + s * _DEG_ROWS_SUB
    pltpu.sync_copy(idx_hbm.at[pl.ds(row0, _DEG_ROWS_SUB)], idx_blk)
    plsc.subcore_barrier()

    def body(g, _):
        for k in range(_DEG_FIRE):
            pltpu.async_copy(ones_v, hist_sp.at[idx_blk.at[g * _DEG_FIRE + k]],
                             sem, add=True)
        for k in range(_DEG_FIRE):
            pltpu.make_async_copy(ones_v, hist_sp.at[idx_blk.at[0]], sem).wait()
        return 0

    lax.fori_loop(0, _DEG_ROWS_SUB // _DEG_FIRE, body, 0)
    plsc.subcore_barrier()
    pltpu.sync_copy(hist_sp.at[pl.ds(s * _HIST_PER_SUB, _HIST_PER_SUB)],
                    out_hbm.at[c, pl.ds(s * _HIST_PER_SUB, _HIST_PER_SUB)])


def _sc_degrees(idx_blocks):
    mesh = plsc.VectorSubcoreMesh(core_axis_name="c", subcore_axis_name="s",
                                  num_cores=NC, num_subcores=NS)
    return pl.kernel(
        _sc_degree_body,
        out_type=jax.ShapeDtypeStruct((NC, HIST), jnp.float32),
        mesh=mesh,
        scratch_types=[
            pltpu.VMEM_SHARED((HIST,), jnp.float32),
            pltpu.VMEM((_DEG_ROWS_SUB, _DEG_C), jnp.int32),
            pltpu.VMEM((_DEG_C,), jnp.float32),
            pltpu.VMEM((_HIST_PER_SUB,), jnp.float32),
            pltpu.SemaphoreType.DMA,
        ],
    )(idx_blocks)


_AGG_C = 120
_AGG_CHUNKS = 167
N_PAD = 10112
_ROWS_PER_SUB = N_PAD // NS
_ZBLK = 8


def _sc_agg_body(table_hbm, pidx_hbm, out_hbm,
                 acc_sp, pidx, rows,
                 g0, g1, g2, s0, s1, s2, i0, i1, i2):
    c = lax.axis_index("c")
    s = lax.axis_index("s")
    zeros16 = jnp.zeros((L,), jnp.float32)
    for r in range(_ZBLK):
        for k in range(H // L):
            rows[0, r, pl.ds(k * L, L)] = zeros16

    row0 = s * _ROWS_PER_SUB
    zsrc = rows.at[0].at[pl.ds(0, _ZBLK)]

    def zbody(i, _):
        pltpu.sync_copy(zsrc, acc_sp.at[pl.ds(row0 + i * _ZBLK, _ZBLK)])
        return 0

    lax.fori_loop(0, _ROWS_PER_SUB // _ZBLK, zbody, 0)
    plsc.subcore_barrier()

    gsems = (g0, g1, g2)
    ssems = (s0, s1, s2)
    isems = (i0, i1, i2)

    def idx_start(j, b):
        pltpu.async_copy(pidx_hbm.at[c, s, j], pidx.at[b], isems[b])

    def idx_wait(j, b):
        pltpu.make_async_copy(pidx_hbm.at[c, s, j], pidx.at[b],
                              isems[b]).wait()

    def gather_start(b):
        pltpu.async_copy(table_hbm.at[pidx.at[b, 0]], rows.at[b], gsems[b])

    def gather_wait(b):
        pltpu.make_async_copy(table_hbm.at[pidx.at[b, 0]], rows.at[b],
                              gsems[b]).wait()

    def scat_start(b):
        pltpu.async_copy(rows.at[b], acc_sp.at[pidx.at[b, 1]], ssems[b],
                         add=True)

    def scat_wait(b):
        pltpu.make_async_copy(rows.at[b], acc_sp.at[pidx.at[b, 1]],
                              ssems[b]).wait()

    idx_start(0, 0)
    idx_start(1, 1)
    idx_wait(0, 0)
    gather_start(0)

    def group(g, _):
        for b in range(3):
            bn = (b + 1) % 3
            bp = (b - 1) % 3
            j = 3 * g + b
            nxt = j + 1

            @pl.when(nxt < _AGG_CHUNKS)
            def _():
                idx_wait(nxt, bn)
                gather_start(bn)
            gather_wait(b)
            scat_start(b)

            @pl.when(j >= 1)
            def _():
                scat_wait(bp)

            @pl.when(nxt + 1 < _AGG_CHUNKS)
            def _():
                idx_start(nxt + 1, bp)
        return 0

    lax.fori_loop(0, (_AGG_CHUNKS - 2) // 3, group, 0)
    idx_wait(_AGG_CHUNKS - 1, 1)
    gather_start(1)
    gather_wait(0)
    scat_start(0)
    scat_wait(2)
    gather_wait(1)
    scat_start(1)
    scat_wait(0)
    scat_wait(1)

    plsc.subcore_barrier()
    pltpu.sync_copy(acc_sp.at[pl.ds(row0, _ROWS_PER_SUB)],
                    out_hbm.at[c, pl.ds(row0, _ROWS_PER_SUB)])


def _sc_aggregate(table, pair_blocks):
    mesh = plsc.VectorSubcoreMesh(core_axis_name="c", subcore_axis_name="s",
                                  num_cores=NC, num_subcores=NS)
    return pl.kernel(
        _sc_agg_body,
        out_type=jax.ShapeDtypeStruct((NC, N_PAD, H), jnp.float32),
        mesh=mesh,
        scratch_types=[
            pltpu.VMEM_SHARED((N_PAD, H), jnp.float32),
            pltpu.VMEM((3, 2, _AGG_C), jnp.int32),
            pltpu.VMEM((3, _AGG_C, H), jnp.float32),
        ] + [pltpu.SemaphoreType.DMA] * 9,
    )(table, pair_blocks)


_BM = 1000


def _rows_call(body, n_out, *args):
    in_specs = []
    for a in args:
        if a.shape[0] == N_DRUG:
            in_specs.append(pl.BlockSpec((_BM, a.shape[1]), lambda i: (i, 0)))
        else:
            in_specs.append(pl.BlockSpec(a.shape, lambda i: (0, 0)))
    outs = [jax.ShapeDtypeStruct((N_DRUG, H), jnp.float32)] * n_out
    return pl.pallas_call(
        body,
        grid=(N_DRUG // _BM,),
        in_specs=in_specs,
        out_specs=[pl.BlockSpec((_BM, H), lambda i: (i, 0))] * n_out,
        out_shape=outs,
    )(*args)


def _proj_body(x_ref, W_ref, b_ref, degs_ref, h_ref, tab_ref):
    h = lax.dot_general(x_ref[...], W_ref[...], (((1,), (1,)), ((), ())),
                        preferred_element_type=jnp.float32) + b_ref[...]
    h_ref[...] = h
    tab_ref[...] = h * lax.rsqrt(jnp.maximum(degs_ref[...], 1.0))


def _post_body(agg_ref, degd_ref, degs_ref, W_ref, b_ref, gam_ref, bet_ref,
               a_ref, h_ref, tab_ref):
    x = agg_ref[...] * lax.rsqrt(jnp.maximum(degd_ref[...], 1.0))
    v = lax.dot_general(x, W_ref[...], (((1,), (1,)), ((), ())),
                        preferred_element_type=jnp.float32) + b_ref[...]
    v = gam_ref[...] * v + bet_ref[...]
    h = jnp.where(v >= 0, v, a_ref[...] * v)
    h_ref[...] = h
    tab_ref[...] = h * lax.rsqrt(jnp.maximum(degs_ref[...], 1.0))


def _att_score_body(h0_ref, h1_ref, h2_ref, W1_ref, b1_ref, w2_ref, o_ref):
    step = pl.program_id(0)

    @pl.when(step == 0)
    def _():
        o_ref[...] = jnp.zeros_like(o_ref)

    lane = lax.broadcasted_iota(jnp.int32, (1, H), 1)
    acc = o_ref[...]
    for l, h_ref in enumerate((h0_ref, h1_ref, h2_ref)):
        t = jnp.tanh(lax.dot_general(h_ref[...], W1_ref[...],
                                     (((1,), (1,)), ((), ())),
                                     preferred_element_type=jnp.float32)
                     + b1_ref[...])
        sl = jnp.sum(t * w2_ref[...])
        acc = acc + jnp.where(lane == l, sl, 0.0)
    o_ref[...] = acc


def _att_scores(h0, h1, h2, W1, b1, w2):
    return pl.pallas_call(
        _att_score_body,
        grid=(N_DRUG // _BM,),
        in_specs=[pl.BlockSpec((_BM, H), lambda i: (i, 0))] * 3
        + [pl.BlockSpec((H, H), lambda i: (0, 0)),
           pl.BlockSpec((1, H), lambda i: (0, 0)),
           pl.BlockSpec((1, H), lambda i: (0, 0))],
        out_specs=pl.BlockSpec((1, H), lambda i: (0, 0)),
        out_shape=jax.ShapeDtypeStruct((1, H), jnp.float32),
    )(h0, h1, h2, W1, b1, w2)


def _combine_body(h0_ref, h1_ref, h2_ref, beta_ref, wr_ref, wd_ref, o_ref):
    f = (h0_ref[...] * beta_ref[0, 0] + h1_ref[...] * beta_ref[0, 1]
         + h2_ref[...] * beta_ref[0, 2])
    m = lax.dot_general(wr_ref[...], wd_ref[...], (((0,), (0,)), ((), ())),
                        preferred_element_type=jnp.float32)
    o_ref[...] = jnp.dot(f, m,
                         preferred_element_type=jnp.float32).astype(jnp.bfloat16)


def _combine(h0, h1, h2, beta_pad, wr, wd):
    return pl.pallas_call(
        _combine_body,
        grid=(N_DRUG // _BM,),
        in_specs=[pl.BlockSpec((_BM, H), lambda i: (i, 0))] * 3
        + [pl.BlockSpec((1, H), lambda i: (0, 0)),
           pl.BlockSpec((H, H), lambda i: (0, 0)),
           pl.BlockSpec((H, H), lambda i: (0, 0))],
        out_specs=pl.BlockSpec((_BM, H), lambda i: (i, 0)),
        out_shape=jax.ShapeDtypeStruct((N_DRUG, H), jnp.bfloat16),
    )(h0, h1, h2, beta_pad, wr, wd)


def _decoder_matmul_kernel(a_ref, b_ref, o_ref):
    o_ref[...] = lax.dot_general(
        a_ref[...], b_ref[...], (((1,), (1,)), ((), ())),
        preferred_element_type=jnp.float32)


def _decoder_matmul(a, b, bm=512, bn=512):
    m, k = a.shape
    n = b.shape[0]
    grid = (pl.cdiv(m, bm), pl.cdiv(n, bn))
    return pl.pallas_call(
        _decoder_matmul_kernel,
        grid=grid,
        in_specs=[
            pl.BlockSpec((bm, k), lambda i, j: (i, 0)),
            pl.BlockSpec((bn, k), lambda i, j: (j, 0)),
        ],
        out_specs=pl.BlockSpec((bm, bn), lambda i, j: (i, j)),
        out_shape=jax.ShapeDtypeStruct((m, n), jnp.float32),
    )(a, b)


def _edge_blocks(idx, offset, pad_value):
    per_sub = E // NS
    pad = _AGG_CHUNKS * _AGG_C - per_sub
    blk = idx.reshape(NS, per_sub) + offset
    blk = jnp.pad(blk, ((0, 0), (0, pad)), constant_values=pad_value)
    return blk.reshape(NS, _AGG_CHUNKS, _AGG_C)


def kernel(x_drug, x_disease, edge_dr2di, edge_di2dr,
           W_drug_lin, b_drug_lin, W_dis_lin, b_dis_lin,
           e1_W_dr2di, e1_b_dr2di, e1_W_di2dr, e1_b_di2dr, e1_gamma, e1_beta, e1_prelu,
           e2_W_dr2di, e2_b_dr2di, e2_W_di2dr, e2_b_di2dr, e2_gamma, e2_beta, e2_prelu,
           att_dr_W1, att_dr_b1, att_dr_w2, att_di_W1, att_di_b1, att_di_w2,
           W_R, W_D):
    pair_blocks = jnp.stack([
        jnp.stack([_edge_blocks(edge_dr2di[0], 0, 0),
                   _edge_blocks(edge_dr2di[1], 0, N_PAD - 8)], axis=2),
        jnp.stack([_edge_blocks(edge_di2dr[0], N_DRUG, 0),
                   _edge_blocks(edge_di2dr[1], 0, N_PAD - 8)], axis=2),
    ])

    src_all = jnp.concatenate([edge_dr2di[0], edge_di2dr[0] + N_DRUG])
    dst_off = jnp.concatenate([edge_dr2di[1], edge_di2dr[1] + N_DIS])
    deg_idx = jnp.concatenate([src_all, dst_off])
    deg_pad = _DEG_ROWS * _DEG_C - deg_idx.shape[0]
    deg_idx = jnp.pad(deg_idx, (0, deg_pad), constant_values=N_ALL)
    deg_blocks = deg_idx.reshape(_DEG_ROWS, _DEG_C)

    hists = _sc_degrees(deg_blocks)
    deg_s_dr = hists[0, :N_DRUG].reshape(-1, 1)
    deg_s_di = hists[0, N_DRUG:N_ALL].reshape(-1, 1)
    deg_d_di = hists[1, :N_DIS].reshape(-1, 1)
    deg_d_dr = hists[1, N_DIS:N_ALL].reshape(-1, 1)

    h_dr0, tab_dr = _rows_call(_proj_body, 2, x_drug, W_drug_lin,
                               b_drug_lin.reshape(1, -1), deg_s_dr)
    h_di0, tab_di = _rows_call(_proj_body, 2, x_disease, W_dis_lin,
                               b_dis_lin.reshape(1, -1), deg_s_di)

    g1 = e1_gamma.reshape(1, -1)
    be1 = e1_beta.reshape(1, -1)
    a1 = jnp.broadcast_to(e1_prelu.reshape(1, 1), (1, H))
    g2 = e2_gamma.reshape(1, -1)
    be2 = e2_beta.reshape(1, -1)
    a2 = jnp.broadcast_to(e2_prelu.reshape(1, 1), (1, H))

    table1 = jnp.concatenate([tab_dr, tab_di])
    agg1 = _sc_aggregate(table1, pair_blocks)
    h_di1, tab_di1 = _rows_call(_post_body, 2, agg1[0, :N_DIS], deg_d_di,
                                deg_s_di, e1_W_dr2di,
                                e1_b_dr2di.reshape(1, -1), g1, be1, a1)
    h_dr1, tab_dr1 = _rows_call(_post_body, 2, agg1[1, :N_DRUG], deg_d_dr,
                                deg_s_dr, e1_W_di2dr,
                                e1_b_di2dr.reshape(1, -1), g1, be1, a1)

    table2 = jnp.concatenate([tab_dr1, tab_di1])
    agg2 = _sc_aggregate(table2, pair_blocks)
    h_di2, _ = _rows_call(_post_body, 2, agg2[0, :N_DIS], deg_d_di,
                          deg_s_di, e2_W_dr2di,
                          e2_b_dr2di.reshape(1, -1), g2, be2, a2)
    h_dr2, _ = _rows_call(_post_body, 2, agg2[1, :N_DRUG], deg_d_dr,
                          deg_s_dr, e2_W_di2dr,
                          e2_b_di2dr.reshape(1, -1), g2, be2, a2)

    s_dr = _att_scores(h_dr0, h_dr1, h_dr2, att_dr_W1,
                       att_dr_b1.reshape(1, -1), att_dr_w2.reshape(1, -1))
    s_di = _att_scores(h_di0, h_di1, h_di2, att_di_W1,
                       att_di_b1.reshape(1, -1), att_di_w2.reshape(1, -1))
    beta_dr = jnp.pad(jax.nn.softmax(s_dr[0, :3] / N_DRUG), (0, H - 3))
    beta_di = jnp.pad(jax.nn.softmax(s_di[0, :3] / N_DIS), (0, H - 3))

    eye = jnp.eye(H, dtype=jnp.float32)
    a_mat = _combine(h_dr0, h_dr1, h_dr2, beta_dr.reshape(1, H), W_R, W_D)
    b_mat = _combine(h_di0, h_di1, h_di2, beta_di.reshape(1, H), eye, eye)
    return _decoder_matmul(a_mat, b_mat)


# speedup vs baseline: 1.3364x; 1.2296x over previous
import functools

import jax
import jax.numpy as jnp
from jax import lax
from jax.experimental import pallas as pl
from jax.experimental.pallas import tpu as pltpu
from jax.experimental.pallas import tpu_sc as plsc

N_DRUG = 10000
N_DIS = 10000
N_ALL = N_DRUG + N_DIS
E = 320000
H = 128

NC = 2
NS = 16
L = 16

HIST = 20480
_DEG_C = 80
_DEG_ROWS = 16384
_DEG_ROWS_SUB = _DEG_ROWS // (NC * NS)
_DEG_FIRE = 8
_HIST_PER_SUB = HIST // NS


def _sc_degree_body(idx_hbm, out_hbm, hist_sp, idx_blk, ones_v, zero_v, sem):
    c = lax.axis_index("c")
    s = lax.axis_index("s")
    zeros16 = jnp.zeros((L,), jnp.float32)
    ones16 = jnp.ones((L,), jnp.float32)
    def fill_zero(i, _):
        zero_v[pl.ds(i * L, L)] = zeros16
        return 0
    lax.fori_loop(0, _HIST_PER_SUB // L, fill_zero, 0)
    for k in range(_DEG_C // L):
        ones_v[pl.ds(k * L, L)] = ones16
    pltpu.sync_copy(zero_v, hist_sp.at[pl.ds(s * _HIST_PER_SUB, _HIST_PER_SUB)])

    row0 = c * (_DEG_ROWS // 2) + s * _DEG_ROWS_SUB
    pltpu.sync_copy(idx_hbm.at[pl.ds(row0, _DEG_ROWS_SUB)], idx_blk)
    plsc.subcore_barrier()

    def body(g, _):
        for k in range(_DEG_FIRE):
            pltpu.async_copy(ones_v, hist_sp.at[idx_blk.at[g * _DEG_FIRE + k]],
                             sem, add=True)
        for k in range(_DEG_FIRE):
            pltpu.make_async_copy(ones_v, hist_sp.at[idx_blk.at[0]], sem).wait()
        return 0

    lax.fori_loop(0, _DEG_ROWS_SUB // _DEG_FIRE, body, 0)
    plsc.subcore_barrier()
    pltpu.sync_copy(hist_sp.at[pl.ds(s * _HIST_PER_SUB, _HIST_PER_SUB)],
                    out_hbm.at[c, pl.ds(s * _HIST_PER_SUB, _HIST_PER_SUB)])


def _sc_degrees(idx_blocks):
    mesh = plsc.VectorSubcoreMesh(core_axis_name="c", subcore_axis_name="s",
                                  num_cores=NC, num_subcores=NS)
    return pl.kernel(
        _sc_degree_body,
        out_type=jax.ShapeDtypeStruct((NC, HIST), jnp.float32),
        mesh=mesh,
        scratch_types=[
            pltpu.VMEM_SHARED((HIST,), jnp.float32),
            pltpu.VMEM((_DEG_ROWS_SUB, _DEG_C), jnp.int32),
            pltpu.VMEM((_DEG_C,), jnp.float32),
            pltpu.VMEM((_HIST_PER_SUB,), jnp.float32),
            pltpu.SemaphoreType.DMA,
        ],
    )(idx_blocks)


_AGG_C = 120
_AGG_CHUNKS = 167
N_PAD = 10112
_ROWS_PER_SUB = N_PAD // NS
_ZBLK = 8


def _sc_agg_body(table_hbm, pidx_hbm, out_hbm,
                 acc_sp, pidx, rows,
                 g0, g1, g2, s0, s1, s2, i0, i1, i2):
    c = lax.axis_index("c")
    s = lax.axis_index("s")
    zeros16 = jnp.zeros((L,), jnp.float32)
    for r in range(_ZBLK):
        for k in range(H // L):
            rows[0, r, pl.ds(k * L, L)] = zeros16

    row0 = s * _ROWS_PER_SUB
    zsrc = rows.at[0].at[pl.ds(0, _ZBLK)]

    def zbody(i, _):
        pltpu.sync_copy(zsrc, acc_sp.at[pl.ds(row0 + i * _ZBLK, _ZBLK)])
        return 0

    lax.fori_loop(0, _ROWS_PER_SUB // _ZBLK, zbody, 0)
    plsc.subcore_barrier()

    gsems = (g0, g1, g2)
    ssems = (s0, s1, s2)
    isems = (i0, i1, i2)

    def idx_start(j, b):
        pltpu.async_copy(pidx_hbm.at[c, s, j], pidx.at[b], isems[b])

    def idx_wait(j, b):
        pltpu.make_async_copy(pidx_hbm.at[c, s, j], pidx.at[b],
                              isems[b]).wait()

    def gather_start(b):
        pltpu.async_copy(table_hbm.at[pidx.at[b, 0]], rows.at[b], gsems[b])

    def gather_wait(b):
        pltpu.make_async_copy(table_hbm.at[pidx.at[b, 0]], rows.at[b],
                              gsems[b]).wait()

    def scat_start(b):
        pltpu.async_copy(rows.at[b], acc_sp.at[pidx.at[b, 1]], ssems[b],
                         add=True)

    def scat_wait(b):
        pltpu.make_async_copy(rows.at[b], acc_sp.at[pidx.at[b, 1]],
                              ssems[b]).wait()

    idx_start(0, 0)
    idx_start(1, 1)
    idx_wait(0, 0)
    gather_start(0)

    def group(g, _):
        for b in range(3):
            bn = (b + 1) % 3
            bp = (b - 1) % 3
            j = 3 * g + b
            nxt = j + 1

            @pl.when(nxt < _AGG_CHUNKS)
            def _():
                idx_wait(nxt, bn)
                gather_start(bn)
            gather_wait(b)
            scat_start(b)

            @pl.when(j >= 1)
            def _():
                scat_wait(bp)

            @pl.when(nxt + 1 < _AGG_CHUNKS)
            def _():
                idx_start(nxt + 1, bp)
        return 0

    lax.fori_loop(0, (_AGG_CHUNKS - 2) // 3, group, 0)
    idx_wait(_AGG_CHUNKS - 1, 1)
    gather_start(1)
    gather_wait(0)
    scat_start(0)
    scat_wait(2)
    gather_wait(1)
    scat_start(1)
    scat_wait(0)
    scat_wait(1)

    plsc.subcore_barrier()
    pltpu.sync_copy(acc_sp.at[pl.ds(row0, _ROWS_PER_SUB)],
                    out_hbm.at[c, pl.ds(row0, _ROWS_PER_SUB)])


def _sc_aggregate(table, pair_blocks):
    mesh = plsc.VectorSubcoreMesh(core_axis_name="c", subcore_axis_name="s",
                                  num_cores=NC, num_subcores=NS)
    return pl.kernel(
        _sc_agg_body,
        out_type=jax.ShapeDtypeStruct((NC, N_PAD, H), jnp.float32),
        mesh=mesh,
        scratch_types=[
            pltpu.VMEM_SHARED((N_PAD, H), jnp.float32),
            pltpu.VMEM((3, 2, _AGG_C), jnp.int32),
            pltpu.VMEM((3, _AGG_C, H), jnp.float32),
        ] + [pltpu.SemaphoreType.DMA] * 9,
    )(table, pair_blocks)


_BM = 2000


def _rows_call(body, n_out, *args):
    in_specs = []
    for a in args:
        if a.shape[0] == N_DRUG:
            in_specs.append(pl.BlockSpec((_BM, a.shape[1]), lambda i: (i, 0)))
        else:
            in_specs.append(pl.BlockSpec(a.shape, lambda i: (0, 0)))
    outs = [jax.ShapeDtypeStruct((N_DRUG, H), jnp.float32)] * n_out
    return pl.pallas_call(
        body,
        grid=(N_DRUG // _BM,),
        in_specs=in_specs,
        out_specs=[pl.BlockSpec((_BM, H), lambda i: (i, 0))] * n_out,
        out_shape=outs,
    )(*args)


def _proj_body(x_ref, W_ref, b_ref, degs_ref, h_ref, tab_ref):
    h = lax.dot_general(x_ref[...], W_ref[...], (((1,), (1,)), ((), ())),
                        preferred_element_type=jnp.float32) + b_ref[...]
    h_ref[...] = h
    tab_ref[...] = h * lax.rsqrt(jnp.maximum(degs_ref[...], 1.0))


def _post_body(agg_ref, degd_ref, degs_ref, W_ref, b_ref, gam_ref, bet_ref,
               a_ref, h_ref, tab_ref):
    x = agg_ref[...] * lax.rsqrt(jnp.maximum(degd_ref[...], 1.0))
    v = lax.dot_general(x, W_ref[...], (((1,), (1,)), ((), ())),
                        preferred_element_type=jnp.float32) + b_ref[...]
    v = gam_ref[...] * v + bet_ref[...]
    h = jnp.where(v >= 0, v, a_ref[...] * v)
    h_ref[...] = h
    tab_ref[...] = h * lax.rsqrt(jnp.maximum(degs_ref[...], 1.0))


def _att_score_body(h0_ref, h1_ref, h2_ref, W1_ref, b1_ref, w2_ref, o_ref):
    step = pl.program_id(0)

    @pl.when(step == 0)
    def _():
        o_ref[...] = jnp.zeros_like(o_ref)

    lane = lax.broadcasted_iota(jnp.int32, (1, H), 1)
    acc = o_ref[...]
    for l, h_ref in enumerate((h0_ref, h1_ref, h2_ref)):
        t = jnp.tanh(lax.dot_general(h_ref[...], W1_ref[...],
                                     (((1,), (1,)), ((), ())),
                                     preferred_element_type=jnp.float32)
                     + b1_ref[...])
        sl = jnp.sum(t * w2_ref[...])
        acc = acc + jnp.where(lane == l, sl, 0.0)
    o_ref[...] = acc


def _att_scores(h0, h1, h2, W1, b1, w2):
    return pl.pallas_call(
        _att_score_body,
        grid=(N_DRUG // _BM,),
        in_specs=[pl.BlockSpec((_BM, H), lambda i: (i, 0))] * 3
        + [pl.BlockSpec((H, H), lambda i: (0, 0)),
           pl.BlockSpec((1, H), lambda i: (0, 0)),
           pl.BlockSpec((1, H), lambda i: (0, 0))],
        out_specs=pl.BlockSpec((1, H), lambda i: (0, 0)),
        out_shape=jax.ShapeDtypeStruct((1, H), jnp.float32),
    )(h0, h1, h2, W1, b1, w2)


def _combine_body(h0_ref, h1_ref, h2_ref, beta_ref, wr_ref, wd_ref, o_ref):
    f = (h0_ref[...] * beta_ref[0, 0] + h1_ref[...] * beta_ref[0, 1]
         + h2_ref[...] * beta_ref[0, 2])
    m = lax.dot_general(wr_ref[...], wd_ref[...], (((0,), (0,)), ((), ())),
                        preferred_element_type=jnp.float32)
    o_ref[...] = jnp.dot(f, m,
                         preferred_element_type=jnp.float32).astype(jnp.bfloat16)


def _combine(h0, h1, h2, beta_pad, wr, wd):
    return pl.pallas_call(
        _combine_body,
        grid=(N_DRUG // _BM,),
        in_specs=[pl.BlockSpec((_BM, H), lambda i: (i, 0))] * 3
        + [pl.BlockSpec((1, H), lambda i: (0, 0)),
           pl.BlockSpec((H, H), lambda i: (0, 0)),
           pl.BlockSpec((H, H), lambda i: (0, 0))],
        out_specs=pl.BlockSpec((_BM, H), lambda i: (i, 0)),
        out_shape=jax.ShapeDtypeStruct((N_DRUG, H), jnp.bfloat16),
    )(h0, h1, h2, beta_pad, wr, wd)


def _decoder_matmul_kernel(a_ref, b_ref, o_ref):
    o_ref[...] = lax.dot_general(
        a_ref[...], b_ref[...], (((1,), (1,)), ((), ())),
        preferred_element_type=jnp.float32)


def _decoder_matmul(a, b, bm=1024, bn=1024):
    m, k = a.shape
    n = b.shape[0]
    grid = (pl.cdiv(m, bm), pl.cdiv(n, bn))
    return pl.pallas_call(
        _decoder_matmul_kernel,
        grid=grid,
        in_specs=[
            pl.BlockSpec((bm, k), lambda i, j: (i, 0)),
            pl.BlockSpec((bn, k), lambda i, j: (j, 0)),
        ],
        out_specs=pl.BlockSpec((bm, bn), lambda i, j: (i, j)),
        out_shape=jax.ShapeDtypeStruct((m, n), jnp.float32),
    )(a, b)


def _edge_blocks(idx, offset, pad_value):
    per_sub = E // NS
    pad = _AGG_CHUNKS * _AGG_C - per_sub
    blk = idx.reshape(NS, per_sub) + offset
    blk = jnp.pad(blk, ((0, 0), (0, pad)), constant_values=pad_value)
    return blk.reshape(NS, _AGG_CHUNKS, _AGG_C)


def kernel(x_drug, x_disease, edge_dr2di, edge_di2dr,
           W_drug_lin, b_drug_lin, W_dis_lin, b_dis_lin,
           e1_W_dr2di, e1_b_dr2di, e1_W_di2dr, e1_b_di2dr, e1_gamma, e1_beta, e1_prelu,
           e2_W_dr2di, e2_b_dr2di, e2_W_di2dr, e2_b_di2dr, e2_gamma, e2_beta, e2_prelu,
           att_dr_W1, att_dr_b1, att_dr_w2, att_di_W1, att_di_b1, att_di_w2,
           W_R, W_D):
    pair_blocks = jnp.stack([
        jnp.stack([_edge_blocks(edge_dr2di[0], 0, 0),
                   _edge_blocks(edge_dr2di[1], 0, N_PAD - 8)], axis=2),
        jnp.stack([_edge_blocks(edge_di2dr[0], N_DRUG, 0),
                   _edge_blocks(edge_di2dr[1], 0, N_PAD - 8)], axis=2),
    ])

    src_all = jnp.concatenate([edge_dr2di[0], edge_di2dr[0] + N_DRUG])
    dst_off = jnp.concatenate([edge_dr2di[1], edge_di2dr[1] + N_DIS])
    deg_idx = jnp.concatenate([src_all, dst_off])
    deg_pad = _DEG_ROWS * _DEG_C - deg_idx.shape[0]
    deg_idx = jnp.pad(deg_idx, (0, deg_pad), constant_values=N_ALL)
    deg_blocks = deg_idx.reshape(_DEG_ROWS, _DEG_C)

    hists = _sc_degrees(deg_blocks)
    deg_s_dr = hists[0, :N_DRUG].reshape(-1, 1)
    deg_s_di = hists[0, N_DRUG:N_ALL].reshape(-1, 1)
    deg_d_di = hists[1, :N_DIS].reshape(-1, 1)
    deg_d_dr = hists[1, N_DIS:N_ALL].reshape(-1, 1)

    h_dr0, tab_dr = _rows_call(_proj_body, 2, x_drug, W_drug_lin,
                               b_drug_lin.reshape(1, -1), deg_s_dr)
    h_di0, tab_di = _rows_call(_proj_body, 2, x_disease, W_dis_lin,
                               b_dis_lin.reshape(1, -1), deg_s_di)

    g1 = e1_gamma.reshape(1, -1)
    be1 = e1_beta.reshape(1, -1)
    a1 = jnp.broadcast_to(e1_prelu.reshape(1, 1), (1, H))
    g2 = e2_gamma.reshape(1, -1)
    be2 = e2_beta.reshape(1, -1)
    a2 = jnp.broadcast_to(e2_prelu.reshape(1, 1), (1, H))

    table1 = jnp.concatenate([tab_dr, tab_di])
    agg1 = _sc_aggregate(table1, pair_blocks)
    h_di1, tab_di1 = _rows_call(_post_body, 2, agg1[0, :N_DIS], deg_d_di,
                                deg_s_di, e1_W_dr2di,
                                e1_b_dr2di.reshape(1, -1), g1, be1, a1)
    h_dr1, tab_dr1 = _rows_call(_post_body, 2, agg1[1, :N_DRUG], deg_d_dr,
                                deg_s_dr, e1_W_di2dr,
                                e1_b_di2dr.reshape(1, -1), g1, be1, a1)

    table2 = jnp.concatenate([tab_dr1, tab_di1])
    agg2 = _sc_aggregate(table2, pair_blocks)
    h_di2, _ = _rows_call(_post_body, 2, agg2[0, :N_DIS], deg_d_di,
                          deg_s_di, e2_W_dr2di,
                          e2_b_dr2di.reshape(1, -1), g2, be2, a2)
    h_dr2, _ = _rows_call(_post_body, 2, agg2[1, :N_DRUG], deg_d_dr,
                          deg_s_dr, e2_W_di2dr,
                          e2_b_di2dr.reshape(1, -1), g2, be2, a2)

    s_dr = _att_scores(h_dr0, h_dr1, h_dr2, att_dr_W1,
                       att_dr_b1.reshape(1, -1), att_dr_w2.reshape(1, -1))
    s_di = _att_scores(h_di0, h_di1, h_di2, att_di_W1,
                       att_di_b1.reshape(1, -1), att_di_w2.reshape(1, -1))
    beta_dr = jnp.pad(jax.nn.softmax(s_dr[0, :3] / N_DRUG), (0, H - 3))
    beta_di = jnp.pad(jax.nn.softmax(s_di[0, :3] / N_DIS), (0, H - 3))

    eye = jnp.eye(H, dtype=jnp.float32)
    a_mat = _combine(h_dr0, h_dr1, h_dr2, beta_dr.reshape(1, H), W_R, W_D)
    b_mat = _combine(h_di0, h_di1, h_di2, beta_di.reshape(1, H), eye, eye)
    return _decoder_matmul(a_mat, b_mat)


# speedup vs baseline: 1.3881x; 1.0387x over previous
import functools

import jax
import jax.numpy as jnp
from jax import lax
from jax.experimental import pallas as pl
from jax.experimental.pallas import tpu as pltpu
from jax.experimental.pallas import tpu_sc as plsc

N_DRUG = 10000
N_DIS = 10000
N_ALL = N_DRUG + N_DIS
E = 320000
H = 128

NC = 2
NS = 16
L = 16

HIST = 20480
_DEG_C = 80
_DEG_ROWS = 16384
_DEG_ROWS_SUB = _DEG_ROWS // (NC * NS)
_DEG_FIRE = 8
_HIST_PER_SUB = HIST // NS


def _sc_degree_body(idx_hbm, out_hbm, hist_sp, idx_blk, ones_v, zero_v, sem):
    c = lax.axis_index("c")
    s = lax.axis_index("s")
    zeros16 = jnp.zeros((L,), jnp.float32)
    ones16 = jnp.ones((L,), jnp.float32)
    def fill_zero(i, _):
        zero_v[pl.ds(i * L, L)] = zeros16
        return 0
    lax.fori_loop(0, _HIST_PER_SUB // L, fill_zero, 0)
    for k in range(_DEG_C // L):
        ones_v[pl.ds(k * L, L)] = ones16
    pltpu.sync_copy(zero_v, hist_sp.at[pl.ds(s * _HIST_PER_SUB, _HIST_PER_SUB)])

    row0 = c * (_DEG_ROWS // 2) + s * _DEG_ROWS_SUB
    pltpu.sync_copy(idx_hbm.at[pl.ds(row0, _DEG_ROWS_SUB)], idx_blk)
    plsc.subcore_barrier()

    def body(g, _):
        for k in range(_DEG_FIRE):
            pltpu.async_copy(ones_v, hist_sp.at[idx_blk.at[g * _DEG_FIRE + k]],
                             sem, add=True)
        for k in range(_DEG_FIRE):
            pltpu.make_async_copy(ones_v, hist_sp.at[idx_blk.at[0]], sem).wait()
        return 0

    lax.fori_loop(0, _DEG_ROWS_SUB // _DEG_FIRE, body, 0)
    plsc.subcore_barrier()
    pltpu.sync_copy(hist_sp.at[pl.ds(s * _HIST_PER_SUB, _HIST_PER_SUB)],
                    out_hbm.at[c, pl.ds(s * _HIST_PER_SUB, _HIST_PER_SUB)])


def _sc_degrees(idx_blocks):
    mesh = plsc.VectorSubcoreMesh(core_axis_name="c", subcore_axis_name="s",
                                  num_cores=NC, num_subcores=NS)
    return pl.kernel(
        _sc_degree_body,
        out_type=jax.ShapeDtypeStruct((NC, HIST), jnp.float32),
        mesh=mesh,
        scratch_types=[
            pltpu.VMEM_SHARED((HIST,), jnp.float32),
            pltpu.VMEM((_DEG_ROWS_SUB, _DEG_C), jnp.int32),
            pltpu.VMEM((_DEG_C,), jnp.float32),
            pltpu.VMEM((_HIST_PER_SUB,), jnp.float32),
            pltpu.SemaphoreType.DMA,
        ],
    )(idx_blocks)


_AGG_C = 120
_AGG_CHUNKS = 167
N_PAD = 10112
_ROWS_PER_SUB = N_PAD // NS
_ZBLK = 8


def _sc_agg_body(table_hbm, pidx_hbm, out_hbm,
                 acc_sp, pidx, rows,
                 g0, g1, g2, s0, s1, s2, i0, i1, i2):
    c = lax.axis_index("c")
    s = lax.axis_index("s")
    zeros16 = jnp.zeros((L,), jnp.float32)
    for r in range(_ZBLK):
        for k in range(H // L):
            rows[0, r, pl.ds(k * L, L)] = zeros16

    row0 = s * _ROWS_PER_SUB
    zsrc = rows.at[0].at[pl.ds(0, _ZBLK)]

    def zbody(i, _):
        pltpu.sync_copy(zsrc, acc_sp.at[pl.ds(row0 + i * _ZBLK, _ZBLK)])
        return 0

    lax.fori_loop(0, _ROWS_PER_SUB // _ZBLK, zbody, 0)
    plsc.subcore_barrier()

    gsems = (g0, g1, g2)
    ssems = (s0, s1, s2)
    isems = (i0, i1, i2)

    def idx_start(j, b):
        pltpu.async_copy(pidx_hbm.at[c, s, j], pidx.at[b], isems[b])

    def idx_wait(j, b):
        pltpu.make_async_copy(pidx_hbm.at[c, s, j], pidx.at[b],
                              isems[b]).wait()

    def gather_start(b):
        pltpu.async_copy(table_hbm.at[pidx.at[b, 0]], rows.at[b], gsems[b])

    def gather_wait(b):
        pltpu.make_async_copy(table_hbm.at[pidx.at[b, 0]], rows.at[b],
                              gsems[b]).wait()

    def scat_start(b):
        pltpu.async_copy(rows.at[b], acc_sp.at[pidx.at[b, 1]], ssems[b],
                         add=True)

    def scat_wait(b):
        pltpu.make_async_copy(rows.at[b], acc_sp.at[pidx.at[b, 1]],
                              ssems[b]).wait()

    idx_start(0, 0)
    idx_start(1, 1)
    idx_wait(0, 0)
    gather_start(0)

    def group(g, _):
        for b in range(3):
            bn = (b + 1) % 3
            bp = (b - 1) % 3
            j = 3 * g + b
            nxt = j + 1

            @pl.when(nxt < _AGG_CHUNKS)
            def _():
                idx_wait(nxt, bn)
                gather_start(bn)
            gather_wait(b)
            scat_start(b)

            @pl.when(j >= 1)
            def _():
                scat_wait(bp)

            @pl.when(nxt + 1 < _AGG_CHUNKS)
            def _():
                idx_start(nxt + 1, bp)
        return 0

    lax.fori_loop(0, (_AGG_CHUNKS - 2) // 3, group, 0)
    idx_wait(_AGG_CHUNKS - 1, 1)
    gather_start(1)
    gather_wait(0)
    scat_start(0)
    scat_wait(2)
    gather_wait(1)
    scat_start(1)
    scat_wait(0)
    scat_wait(1)

    plsc.subcore_barrier()
    pltpu.sync_copy(acc_sp.at[pl.ds(row0, _ROWS_PER_SUB)],
                    out_hbm.at[c, pl.ds(row0, _ROWS_PER_SUB)])


def _sc_aggregate(table, pair_blocks):
    mesh = plsc.VectorSubcoreMesh(core_axis_name="c", subcore_axis_name="s",
                                  num_cores=NC, num_subcores=NS)
    return pl.kernel(
        _sc_agg_body,
        out_type=jax.ShapeDtypeStruct((NC, N_PAD, H), jnp.float32),
        mesh=mesh,
        scratch_types=[
            pltpu.VMEM_SHARED((N_PAD, H), jnp.float32),
            pltpu.VMEM((3, 2, _AGG_C), jnp.int32),
            pltpu.VMEM((3, _AGG_C, H), jnp.float32),
        ] + [pltpu.SemaphoreType.DMA] * 9,
    )(table, pair_blocks)


_BM = 2000


def _rows_call(body, n_out, *args):
    in_specs = []
    for a in args:
        if a.shape[0] == N_DRUG:
            in_specs.append(pl.BlockSpec((_BM, a.shape[1]), lambda i: (i, 0)))
        else:
            in_specs.append(pl.BlockSpec(a.shape, lambda i: (0, 0)))
    outs = [jax.ShapeDtypeStruct((N_DRUG, H), jnp.float32)] * n_out
    return pl.pallas_call(
        body,
        grid=(N_DRUG // _BM,),
        in_specs=in_specs,
        out_specs=[pl.BlockSpec((_BM, H), lambda i: (i, 0))] * n_out,
        out_shape=outs,
    )(*args)


def _proj_body(x_ref, W_ref, b_ref, degs_ref, h_ref, tab_ref):
    h = lax.dot_general(x_ref[...], W_ref[...], (((1,), (1,)), ((), ())),
                        preferred_element_type=jnp.float32) + b_ref[...]
    h_ref[...] = h
    tab_ref[...] = h * lax.rsqrt(jnp.maximum(degs_ref[...], 1.0))


def _post_body(agg_ref, degd_ref, degs_ref, W_ref, b_ref, gam_ref, bet_ref,
               a_ref, h_ref, tab_ref):
    x = agg_ref[...] * lax.rsqrt(jnp.maximum(degd_ref[...], 1.0))
    v = lax.dot_general(x, W_ref[...], (((1,), (1,)), ((), ())),
                        preferred_element_type=jnp.float32) + b_ref[...]
    v = gam_ref[...] * v + bet_ref[...]
    h = jnp.where(v >= 0, v, a_ref[...] * v)
    h_ref[...] = h
    tab_ref[...] = h * lax.rsqrt(jnp.maximum(degs_ref[...], 1.0))


def _att_score_body(h0_ref, h1_ref, h2_ref, W1_ref, b1_ref, w2_ref, o_ref):
    step = pl.program_id(0)

    @pl.when(step == 0)
    def _():
        o_ref[...] = jnp.zeros_like(o_ref)

    lane = lax.broadcasted_iota(jnp.int32, (1, H), 1)
    acc = o_ref[...]
    for l, h_ref in enumerate((h0_ref, h1_ref, h2_ref)):
        t = jnp.tanh(lax.dot_general(h_ref[...], W1_ref[...],
                                     (((1,), (1,)), ((), ())),
                                     preferred_element_type=jnp.float32)
                     + b1_ref[...])
        sl = jnp.sum(t * w2_ref[...])
        acc = acc + jnp.where(lane == l, sl, 0.0)
    o_ref[...] = acc


def _att_scores(h0, h1, h2, W1, b1, w2):
    return pl.pallas_call(
        _att_score_body,
        grid=(N_DRUG // _BM,),
        in_specs=[pl.BlockSpec((_BM, H), lambda i: (i, 0))] * 3
        + [pl.BlockSpec((H, H), lambda i: (0, 0)),
           pl.BlockSpec((1, H), lambda i: (0, 0)),
           pl.BlockSpec((1, H), lambda i: (0, 0))],
        out_specs=pl.BlockSpec((1, H), lambda i: (0, 0)),
        out_shape=jax.ShapeDtypeStruct((1, H), jnp.float32),
    )(h0, h1, h2, W1, b1, w2)


def _combine_body(h0_ref, h1_ref, h2_ref, beta_ref, wr_ref, wd_ref, o_ref):
    f = (h0_ref[...] * beta_ref[0, 0] + h1_ref[...] * beta_ref[0, 1]
         + h2_ref[...] * beta_ref[0, 2])
    m = lax.dot_general(wr_ref[...], wd_ref[...], (((0,), (0,)), ((), ())),
                        preferred_element_type=jnp.float32)
    o_ref[...] = jnp.dot(f, m,
                         preferred_element_type=jnp.float32).astype(jnp.bfloat16)


def _combine(h0, h1, h2, beta_pad, wr, wd):
    return pl.pallas_call(
        _combine_body,
        grid=(N_DRUG // _BM,),
        in_specs=[pl.BlockSpec((_BM, H), lambda i: (i, 0))] * 3
        + [pl.BlockSpec((1, H), lambda i: (0, 0)),
           pl.BlockSpec((H, H), lambda i: (0, 0)),
           pl.BlockSpec((H, H), lambda i: (0, 0))],
        out_specs=pl.BlockSpec((_BM, H), lambda i: (i, 0)),
        out_shape=jax.ShapeDtypeStruct((N_DRUG, H), jnp.bfloat16),
    )(h0, h1, h2, beta_pad, wr, wd)


def _decoder_matmul_kernel(a_ref, b_ref, o_ref):
    o_ref[...] = lax.dot_general(
        a_ref[...], b_ref[...], (((1,), (1,)), ((), ())),
        preferred_element_type=jnp.float32)


def _decoder_matmul(a, b, bm=2048, bn=2048):
    m, k = a.shape
    n = b.shape[0]
    grid = (pl.cdiv(m, bm), pl.cdiv(n, bn))
    return pl.pallas_call(
        _decoder_matmul_kernel,
        grid=grid,
        in_specs=[
            pl.BlockSpec((bm, k), lambda i, j: (i, 0)),
            pl.BlockSpec((bn, k), lambda i, j: (j, 0)),
        ],
        out_specs=pl.BlockSpec((bm, bn), lambda i, j: (i, j)),
        out_shape=jax.ShapeDtypeStruct((m, n), jnp.float32),
    )(a, b)


def _edge_blocks(idx, offset, pad_value):
    per_sub = E // NS
    pad = _AGG_CHUNKS * _AGG_C - per_sub
    blk = idx.reshape(NS, per_sub) + offset
    blk = jnp.pad(blk, ((0, 0), (0, pad)), constant_values=pad_value)
    return blk.reshape(NS, _AGG_CHUNKS, _AGG_C)


def kernel(x_drug, x_disease, edge_dr2di, edge_di2dr,
           W_drug_lin, b_drug_lin, W_dis_lin, b_dis_lin,
           e1_W_dr2di, e1_b_dr2di, e1_W_di2dr, e1_b_di2dr, e1_gamma, e1_beta, e1_prelu,
           e2_W_dr2di, e2_b_dr2di, e2_W_di2dr, e2_b_di2dr, e2_gamma, e2_beta, e2_prelu,
           att_dr_W1, att_dr_b1, att_dr_w2, att_di_W1, att_di_b1, att_di_w2,
           W_R, W_D):
    pair_blocks = jnp.stack([
        jnp.stack([_edge_blocks(edge_dr2di[0], 0, 0),
                   _edge_blocks(edge_dr2di[1], 0, N_PAD - 8)], axis=2),
        jnp.stack([_edge_blocks(edge_di2dr[0], N_DRUG, 0),
                   _edge_blocks(edge_di2dr[1], 0, N_PAD - 8)], axis=2),
    ])

    src_all = jnp.concatenate([edge_dr2di[0], edge_di2dr[0] + N_DRUG])
    dst_off = jnp.concatenate([edge_dr2di[1], edge_di2dr[1] + N_DIS])
    deg_idx = jnp.concatenate([src_all, dst_off])
    deg_pad = _DEG_ROWS * _DEG_C - deg_idx.shape[0]
    deg_idx = jnp.pad(deg_idx, (0, deg_pad), constant_values=N_ALL)
    deg_blocks = deg_idx.reshape(_DEG_ROWS, _DEG_C)

    hists = _sc_degrees(deg_blocks)
    deg_s_dr = hists[0, :N_DRUG].reshape(-1, 1)
    deg_s_di = hists[0, N_DRUG:N_ALL].reshape(-1, 1)
    deg_d_di = hists[1, :N_DIS].reshape(-1, 1)
    deg_d_dr = hists[1, N_DIS:N_ALL].reshape(-1, 1)

    h_dr0, tab_dr = _rows_call(_proj_body, 2, x_drug, W_drug_lin,
                               b_drug_lin.reshape(1, -1), deg_s_dr)
    h_di0, tab_di = _rows_call(_proj_body, 2, x_disease, W_dis_lin,
                               b_dis_lin.reshape(1, -1), deg_s_di)

    g1 = e1_gamma.reshape(1, -1)
    be1 = e1_beta.reshape(1, -1)
    a1 = jnp.broadcast_to(e1_prelu.reshape(1, 1), (1, H))
    g2 = e2_gamma.reshape(1, -1)
    be2 = e2_beta.reshape(1, -1)
    a2 = jnp.broadcast_to(e2_prelu.reshape(1, 1), (1, H))

    table1 = jnp.concatenate([tab_dr, tab_di])
    agg1 = _sc_aggregate(table1, pair_blocks)
    h_di1, tab_di1 = _rows_call(_post_body, 2, agg1[0, :N_DIS], deg_d_di,
                                deg_s_di, e1_W_dr2di,
                                e1_b_dr2di.reshape(1, -1), g1, be1, a1)
    h_dr1, tab_dr1 = _rows_call(_post_body, 2, agg1[1, :N_DRUG], deg_d_dr,
                                deg_s_dr, e1_W_di2dr,
                                e1_b_di2dr.reshape(1, -1), g1, be1, a1)

    table2 = jnp.concatenate([tab_dr1, tab_di1])
    agg2 = _sc_aggregate(table2, pair_blocks)
    h_di2, _ = _rows_call(_post_body, 2, agg2[0, :N_DIS], deg_d_di,
                          deg_s_di, e2_W_dr2di,
                          e2_b_dr2di.reshape(1, -1), g2, be2, a2)
    h_dr2, _ = _rows_call(_post_body, 2, agg2[1, :N_DRUG], deg_d_dr,
                          deg_s_dr, e2_W_di2dr,
                          e2_b_di2dr.reshape(1, -1), g2, be2, a2)

    s_dr = _att_scores(h_dr0, h_dr1, h_dr2, att_dr_W1,
                       att_dr_b1.reshape(1, -1), att_dr_w2.reshape(1, -1))
    s_di = _att_scores(h_di0, h_di1, h_di2, att_di_W1,
                       att_di_b1.reshape(1, -1), att_di_w2.reshape(1, -1))
    beta_dr = jnp.pad(jax.nn.softmax(s_dr[0, :3] / N_DRUG), (0, H - 3))
    beta_di = jnp.pad(jax.nn.softmax(s_di[0, :3] / N_DIS), (0, H - 3))

    eye = jnp.eye(H, dtype=jnp.float32)
    a_mat = _combine(h_dr0, h_dr1, h_dr2, beta_dr.reshape(1, H), W_R, W_D)
    b_mat = _combine(h_di0, h_di1, h_di2, beta_di.reshape(1, H), eye, eye)
    return _decoder_matmul(a_mat, b_mat)
